# baseline (speedup 1.0000x reference)
import functools

import jax
import jax.numpy as jnp
from jax import lax
from jax.experimental import pallas as pl
from jax.experimental.pallas import tpu as pltpu
from jax.experimental.pallas import tpu_sc as plsc

_K = 20
_R = 256
_ITERS = 8
_G = 256
_CH = 128

_SC_CORES = 2
_SC_SUBCORES = 16


def _sc_gather(table, idx):
    n, d = table.shape
    nw = _SC_CORES * _SC_SUBCORES
    bpw = n // nw
    mesh = plsc.VectorSubcoreMesh(core_axis_name="c", subcore_axis_name="s")

    @functools.partial(
        pl.kernel, mesh=mesh,
        out_type=jax.ShapeDtypeStruct((n, d), jnp.float32),
        scratch_types=[
            pltpu.VMEM((bpw,), jnp.int32),
            pltpu.VMEM((bpw, d), jnp.float32),
            pltpu.SemaphoreType.DMA,
        ],
    )
    def gather_kernel(table_hbm, idx_hbm, out_hbm, idx_v, rows_v, sem):
        wid = lax.axis_index("s") * _SC_CORES + lax.axis_index("c")
        base = wid * bpw
        pltpu.sync_copy(idx_hbm.at[pl.ds(base, bpw)], idx_v)
        pltpu.async_copy(table_hbm.at[idx_v], rows_v, sem).wait()
        pltpu.sync_copy(rows_v, out_hbm.at[pl.ds(base, bpw)])

    return gather_kernel(table, idx)


def _body(cnt_ref, feats_ref, sf_ref, ofg_ref, obg_ref, fn_ref, sim_ref):
    sf = sf_ref[...]
    n_rows = fn_ref.shape[0]
    cnt_i = cnt_ref[0, 0]

    @pl.when(pl.program_id(0) == 0)
    def _():
        feats = feats_ref[...]
        qn2 = jnp.sum(feats * feats, axis=1, keepdims=True)
        qn_inv = 1.0 / jnp.maximum(jnp.sqrt(qn2), 1e-12)
        fn_ref[...] = feats * qn_inv

    sim_ref[...] = jax.lax.dot_general(
        fn_ref[...], sf, (((1,), (0,)), ((), ())),
        preferred_element_type=jnp.float32)
    sim = sim_ref[...]

    cnt_f = cnt_i.astype(jnp.float32)
    cnt_b = jnp.float32(n_rows) - cnt_f
    kf = jnp.float32(_K)
    n_f = jnp.minimum(kf, cnt_f)
    n_b = jnp.minimum(kf, cnt_b)

    sn2 = jnp.sum(sf * sf, axis=0, keepdims=True)
    sn = jnp.sqrt(sn2)
    neg = jnp.float32(-jnp.inf)

    rowid = lax.broadcasted_iota(jnp.int32, (n_rows, 1), 0)
    infg = rowid < cnt_i

    zg = sim.reshape(_G, n_rows // _G, _R)
    infg_g = infg.reshape(_G, n_rows // _G, 1)
    gmf = jnp.max(jnp.where(infg_g, zg, neg), axis=1)
    gmb = jnp.max(jnp.where(infg_g, neg, zg), axis=1)
    hi_f = jnp.max(gmf, axis=0, keepdims=True)
    hi_b = jnp.max(gmb, axis=0, keepdims=True)
    pos = jnp.float32(jnp.inf)
    lo_f_raw = jnp.min(jnp.where(gmf > jnp.float32(-1e38), gmf, pos),
                       axis=0, keepdims=True)
    lo_b_raw = jnp.min(jnp.where(gmb > jnp.float32(-1e38), gmb, pos),
                       axis=0, keepdims=True)
    grp = jnp.float32(n_rows // _G)
    ok_f = cnt_f >= kf * grp
    ok_b = cnt_b >= kf * grp
    lo_f = jnp.where(ok_f, jnp.maximum(lo_f_raw, -sn), -sn)
    lo_b = jnp.where(ok_b, jnp.maximum(lo_b_raw, -sn), -sn)

    nchunks = n_rows // _CH
    b_idx = jnp.minimum(cnt_i // _CH, nchunks - 1)
    rowid_ch = lax.broadcasted_iota(jnp.int32, (_CH, 1), 0)

    def count_range(lo_c, hi_c, thr):
        def cbody(j, acc):
            slab = sim_ref[pl.ds(j * _CH, _CH), :]
            return acc + jnp.sum((slab >= thr).astype(jnp.float32),
                                 axis=0, keepdims=True)
        return lax.fori_loop(lo_c, hi_c, cbody,
                             jnp.zeros((1, _R), jnp.float32))

    def it(_, st):
        lo_f, hi_f, lo_b, hi_b = st
        mid_f = 0.5 * (lo_f + hi_f)
        mid_b = 0.5 * (lo_b + hi_b)
        cf = count_range(0, b_idx, mid_f)
        cb = count_range(b_idx + 1, nchunks, mid_b)
        slab = sim_ref[pl.ds(b_idx * _CH, _CH), :]
        bpred = (rowid_ch + b_idx * _CH) < cnt_i
        gef = (slab >= mid_f) & bpred
        geb = (slab >= mid_b) & jnp.logical_not(bpred)
        cf = cf + jnp.sum(gef.astype(jnp.float32), axis=0, keepdims=True)
        cb = cb + jnp.sum(geb.astype(jnp.float32), axis=0, keepdims=True)
        pf = cf >= n_f
        pb = cb >= n_b
        lo_f = jnp.where(pf, mid_f, lo_f)
        hi_f = jnp.where(pf, hi_f, mid_f)
        lo_b = jnp.where(pb, mid_b, lo_b)
        hi_b = jnp.where(pb, hi_b, mid_b)
        return lo_f, hi_f, lo_b, hi_b

    lo_f, hi_f, lo_b, hi_b = jax.lax.fori_loop(
        0, _ITERS, it, (lo_f, hi_f, lo_b, hi_b))

    gtf = ((sim > lo_f) & infg).astype(jnp.float32)
    gtb = ((sim > lo_b) & jnp.logical_not(infg)).astype(jnp.float32)
    s_f = jnp.sum(gtf * sim, axis=0, keepdims=True)
    s_b = jnp.sum(gtb * sim, axis=0, keepdims=True)
    cgf = jnp.sum(gtf, axis=0, keepdims=True)
    cgb = jnp.sum(gtb, axis=0, keepdims=True)

    t_f = jnp.where(lo_f > jnp.float32(-1e38), lo_f, 0.0)
    t_b = jnp.where(lo_b > jnp.float32(-1e38), lo_b, 0.0)
    res_f = jnp.where(n_f > 0,
                      (s_f + (n_f - cgf) * t_f) / jnp.maximum(n_f, 1.0), 0.0)
    res_b = jnp.where(n_b > 0,
                      (s_b + (n_b - cgb) * t_b) / jnp.maximum(n_b, 1.0), 0.0)

    sn_inv = 1.0 / jnp.maximum(sn, 1e-12)
    ofg_ref[...] = (res_f * sn_inv).reshape(1, 1, _R)
    obg_ref[...] = (res_b * sn_inv).reshape(1, 1, _R)


@functools.partial(jax.jit, static_argnums=())
def kernel(query_label, color, q_feat, s_feat):
    Hq, Wq = int(q_feat.shape[2]), int(q_feat.shape[3])
    C = int(q_feat.shape[1])
    N = Hq * Wq
    Hs, Ws = int(s_feat.shape[2]), int(s_feat.shape[3])
    M = Hs * Ws

    feats = q_feat.reshape(C, N).T
    sfm = s_feat.reshape(C, M)

    mask = (query_label.reshape(N, 3) == color[None, :]).all(-1)
    iota = jnp.arange(N, dtype=jnp.int32)
    cs = jnp.cumsum(mask.astype(jnp.int32))
    cnt_s = cs[-1]
    dest = jnp.where(mask, cs - 1, cnt_s + iota - cs)
    perm = jnp.zeros((N,), jnp.int32).at[dest].set(iota, unique_indices=True)
    cnt = cnt_s.reshape(1, 1)

    fperm = _sc_gather(feats, perm)

    nblk = M // _R
    out_shape = jax.ShapeDtypeStruct((nblk, 1, _R), jnp.float32)
    ofg, obg = pl.pallas_call(
        _body,
        grid=(nblk,),
        in_specs=[
            pl.BlockSpec(memory_space=pltpu.SMEM),
            pl.BlockSpec((N, C), lambda i: (0, 0)),
            pl.BlockSpec((C, _R), lambda i: (0, i)),
        ],
        out_specs=[
            pl.BlockSpec((1, 1, _R), lambda i: (i, 0, 0)),
            pl.BlockSpec((1, 1, _R), lambda i: (i, 0, 0)),
        ],
        out_shape=[out_shape, out_shape],
        scratch_shapes=[pltpu.VMEM((N, C), jnp.float32),
                        pltpu.VMEM((N, _R), jnp.float32)],
    )(cnt, fperm, sfm)

    return (ofg.reshape(Hs, Ws), obg.reshape(Hs, Ws))


# speedup vs baseline: 1.2957x; 1.2957x over previous
import functools

import jax
import jax.numpy as jnp
from jax import lax
from jax.experimental import pallas as pl
from jax.experimental.pallas import tpu as pltpu
from jax.experimental.pallas import tpu_sc as plsc

_K = 20
_R = 256
_ITERS = 8
_G = 256
_CH = 128

_SC_CORES = 2
_SC_SUBCORES = 16


def _sc_gather(table, idx):
    n, d = table.shape
    nw = _SC_CORES * _SC_SUBCORES
    bpw = n // nw
    mesh = plsc.VectorSubcoreMesh(core_axis_name="c", subcore_axis_name="s")

    @functools.partial(
        pl.kernel, mesh=mesh,
        out_type=jax.ShapeDtypeStruct((n, d), jnp.float32),
        scratch_types=[
            pltpu.VMEM((bpw,), jnp.int32),
            pltpu.VMEM((bpw, d), jnp.float32),
            pltpu.SemaphoreType.DMA,
        ],
    )
    def gather_kernel(table_hbm, idx_hbm, out_hbm, idx_v, rows_v, sem):
        wid = lax.axis_index("s") * _SC_CORES + lax.axis_index("c")
        base = wid * bpw
        pltpu.sync_copy(idx_hbm.at[pl.ds(base, bpw)], idx_v)
        pltpu.async_copy(table_hbm.at[idx_v], rows_v, sem).wait()
        pltpu.sync_copy(rows_v, out_hbm.at[pl.ds(base, bpw)])

    return gather_kernel(table, idx)


def _body(cnt_ref, feats_ref, sf_ref, ofg_ref, obg_ref, fn_ref, sim_ref):
    sf = sf_ref[...]
    n_rows = fn_ref.shape[0]
    cnt_i = cnt_ref[0, 0]

    @pl.when(pl.program_id(0) == 0)
    def _():
        feats = feats_ref[...]
        qn2 = jnp.sum(feats * feats, axis=1, keepdims=True)
        qn_inv = 1.0 / jnp.maximum(jnp.sqrt(qn2), 1e-12)
        fn_ref[...] = feats * qn_inv

    sim_ref[...] = jax.lax.dot_general(
        fn_ref[...], sf, (((1,), (0,)), ((), ())),
        preferred_element_type=jnp.float32)
    sim = sim_ref[...]

    cnt_f = cnt_i.astype(jnp.float32)
    cnt_b = jnp.float32(n_rows) - cnt_f
    kf = jnp.float32(_K)
    n_f = jnp.minimum(kf, cnt_f)
    n_b = jnp.minimum(kf, cnt_b)

    sn2 = jnp.sum(sf * sf, axis=0, keepdims=True)
    sn = jnp.sqrt(sn2)
    neg = jnp.float32(-jnp.inf)

    rowid = lax.broadcasted_iota(jnp.int32, (n_rows, 1), 0)
    infg = rowid < cnt_i

    zg = sim.reshape(_G, n_rows // _G, _R)
    infg_g = infg.reshape(_G, n_rows // _G, 1)
    gmf = jnp.max(jnp.where(infg_g, zg, neg), axis=1)
    gmb = jnp.max(jnp.where(infg_g, neg, zg), axis=1)
    hi_f = jnp.max(gmf, axis=0, keepdims=True)
    hi_b = jnp.max(gmb, axis=0, keepdims=True)
    pos = jnp.float32(jnp.inf)
    lo_f_raw = jnp.min(jnp.where(gmf > jnp.float32(-1e38), gmf, pos),
                       axis=0, keepdims=True)
    lo_b_raw = jnp.min(jnp.where(gmb > jnp.float32(-1e38), gmb, pos),
                       axis=0, keepdims=True)
    grp = jnp.float32(n_rows // _G)
    ok_f = cnt_f >= kf * grp
    ok_b = cnt_b >= kf * grp
    lo_f = jnp.where(ok_f, jnp.maximum(lo_f_raw, -sn), -sn)
    lo_b = jnp.where(ok_b, jnp.maximum(lo_b_raw, -sn), -sn)

    nchunks = n_rows // _CH
    b_idx = jnp.minimum(cnt_i // _CH, nchunks - 1)
    rowid_ch = lax.broadcasted_iota(jnp.int32, (_CH, 1), 0)

    def it(_, st):
        lo_f, hi_f, lo_b, hi_b = st
        mid_f = 0.5 * (lo_f + hi_f)
        mid_b = 0.5 * (lo_b + hi_b)
        cf = jnp.zeros((1, _R), jnp.float32)
        cb = jnp.zeros((1, _R), jnp.float32)
        zero = jnp.zeros((1, _R), jnp.float32)
        for j in range(nchunks):
            thr = jnp.where(j <= b_idx, mid_f, mid_b)
            slab = sim[j * _CH:(j + 1) * _CH, :]
            part = jnp.sum((slab >= thr).astype(jnp.float32),
                           axis=0, keepdims=True)
            cf = cf + jnp.where(j < b_idx, part, zero)
            cb = cb + jnp.where(j > b_idx, part, zero)
        slab = sim_ref[pl.ds(b_idx * _CH, _CH), :]
        bpred = (rowid_ch + b_idx * _CH) < cnt_i
        gef = (slab >= mid_f) & bpred
        geb = (slab >= mid_b) & jnp.logical_not(bpred)
        cf = cf + jnp.sum(gef.astype(jnp.float32), axis=0, keepdims=True)
        cb = cb + jnp.sum(geb.astype(jnp.float32), axis=0, keepdims=True)
        pf = cf >= n_f
        pb = cb >= n_b
        lo_f = jnp.where(pf, mid_f, lo_f)
        hi_f = jnp.where(pf, hi_f, mid_f)
        lo_b = jnp.where(pb, mid_b, lo_b)
        hi_b = jnp.where(pb, hi_b, mid_b)
        return lo_f, hi_f, lo_b, hi_b

    lo_f, hi_f, lo_b, hi_b = jax.lax.fori_loop(
        0, _ITERS, it, (lo_f, hi_f, lo_b, hi_b))

    gtf = ((sim > lo_f) & infg).astype(jnp.float32)
    gtb = ((sim > lo_b) & jnp.logical_not(infg)).astype(jnp.float32)
    s_f = jnp.sum(gtf * sim, axis=0, keepdims=True)
    s_b = jnp.sum(gtb * sim, axis=0, keepdims=True)
    cgf = jnp.sum(gtf, axis=0, keepdims=True)
    cgb = jnp.sum(gtb, axis=0, keepdims=True)

    t_f = jnp.where(lo_f > jnp.float32(-1e38), lo_f, 0.0)
    t_b = jnp.where(lo_b > jnp.float32(-1e38), lo_b, 0.0)
    res_f = jnp.where(n_f > 0,
                      (s_f + (n_f - cgf) * t_f) / jnp.maximum(n_f, 1.0), 0.0)
    res_b = jnp.where(n_b > 0,
                      (s_b + (n_b - cgb) * t_b) / jnp.maximum(n_b, 1.0), 0.0)

    sn_inv = 1.0 / jnp.maximum(sn, 1e-12)
    ofg_ref[...] = (res_f * sn_inv).reshape(1, 1, _R)
    obg_ref[...] = (res_b * sn_inv).reshape(1, 1, _R)


@functools.partial(jax.jit, static_argnums=())
def kernel(query_label, color, q_feat, s_feat):
    Hq, Wq = int(q_feat.shape[2]), int(q_feat.shape[3])
    C = int(q_feat.shape[1])
    N = Hq * Wq
    Hs, Ws = int(s_feat.shape[2]), int(s_feat.shape[3])
    M = Hs * Ws

    feats = q_feat.reshape(C, N).T
    sfm = s_feat.reshape(C, M)

    mask = (query_label.reshape(N, 3) == color[None, :]).all(-1)
    perm = jnp.argsort(jnp.logical_not(mask)).astype(jnp.int32)
    cnt = jnp.sum(mask, dtype=jnp.int32).reshape(1, 1)

    fperm = _sc_gather(feats, perm)

    nblk = M // _R
    out_shape = jax.ShapeDtypeStruct((nblk, 1, _R), jnp.float32)
    ofg, obg = pl.pallas_call(
        _body,
        grid=(nblk,),
        in_specs=[
            pl.BlockSpec(memory_space=pltpu.SMEM),
            pl.BlockSpec((N, C), lambda i: (0, 0)),
            pl.BlockSpec((C, _R), lambda i: (0, i)),
        ],
        out_specs=[
            pl.BlockSpec((1, 1, _R), lambda i: (i, 0, 0)),
            pl.BlockSpec((1, 1, _R), lambda i: (i, 0, 0)),
        ],
        out_shape=[out_shape, out_shape],
        scratch_shapes=[pltpu.VMEM((N, C), jnp.float32),
                        pltpu.VMEM((N, _R), jnp.float32)],
    )(cnt, fperm, sfm)

    return (ofg.reshape(Hs, Ws), obg.reshape(Hs, Ws))


# speedup vs baseline: 1.3651x; 1.0536x over previous
import functools

import jax
import jax.numpy as jnp
from jax import lax
from jax.experimental import pallas as pl
from jax.experimental.pallas import tpu as pltpu
from jax.experimental.pallas import tpu_sc as plsc

_K = 20
_R = 256
_ITERS = 7
_G = 256
_CH = 128

_SC_CORES = 2
_SC_SUBCORES = 16


def _sc_gather(table, idx):
    n, d = table.shape
    nw = _SC_CORES * _SC_SUBCORES
    bpw = n // nw
    mesh = plsc.VectorSubcoreMesh(core_axis_name="c", subcore_axis_name="s")

    @functools.partial(
        pl.kernel, mesh=mesh,
        out_type=jax.ShapeDtypeStruct((n, d), jnp.float32),
        scratch_types=[
            pltpu.VMEM((bpw,), jnp.int32),
            pltpu.VMEM((bpw, d), jnp.float32),
            pltpu.SemaphoreType.DMA,
        ],
    )
    def gather_kernel(table_hbm, idx_hbm, out_hbm, idx_v, rows_v, sem):
        wid = lax.axis_index("s") * _SC_CORES + lax.axis_index("c")
        base = wid * bpw
        pltpu.sync_copy(idx_hbm.at[pl.ds(base, bpw)], idx_v)
        pltpu.async_copy(table_hbm.at[idx_v], rows_v, sem).wait()
        pltpu.sync_copy(rows_v, out_hbm.at[pl.ds(base, bpw)])

    return gather_kernel(table, idx)


def _body(cnt_ref, feats_ref, sf_ref, ofg_ref, obg_ref, fn_ref, sim_ref):
    sf = sf_ref[...]
    n_rows = fn_ref.shape[0]
    cnt_i = cnt_ref[0, 0]

    @pl.when(pl.program_id(0) == 0)
    def _():
        feats = feats_ref[...]
        qn2 = jnp.sum(feats * feats, axis=1, keepdims=True)
        qn_inv = 1.0 / jnp.maximum(jnp.sqrt(qn2), 1e-12)
        fn_ref[...] = feats * qn_inv

    sim_ref[...] = jax.lax.dot_general(
        fn_ref[...], sf, (((1,), (0,)), ((), ())),
        preferred_element_type=jnp.float32)
    sim = sim_ref[...]

    cnt_f = cnt_i.astype(jnp.float32)
    cnt_b = jnp.float32(n_rows) - cnt_f
    kf = jnp.float32(_K)
    n_f = jnp.minimum(kf, cnt_f)
    n_b = jnp.minimum(kf, cnt_b)

    sn2 = jnp.sum(sf * sf, axis=0, keepdims=True)
    sn = jnp.sqrt(sn2)
    neg = jnp.float32(-jnp.inf)

    rowid = lax.broadcasted_iota(jnp.int32, (n_rows, 1), 0)
    infg = rowid < cnt_i

    zg = sim.reshape(_G, n_rows // _G, _R)
    infg_g = infg.reshape(_G, n_rows // _G, 1)
    gmf = jnp.max(jnp.where(infg_g, zg, neg), axis=1)
    gmb = jnp.max(jnp.where(infg_g, neg, zg), axis=1)
    hi_f = jnp.max(gmf, axis=0, keepdims=True)
    hi_b = jnp.max(gmb, axis=0, keepdims=True)
    pos = jnp.float32(jnp.inf)
    lo_f_raw = jnp.min(jnp.where(gmf > jnp.float32(-1e38), gmf, pos),
                       axis=0, keepdims=True)
    lo_b_raw = jnp.min(jnp.where(gmb > jnp.float32(-1e38), gmb, pos),
                       axis=0, keepdims=True)
    grp = jnp.float32(n_rows // _G)
    ok_f = cnt_f >= kf * grp
    ok_b = cnt_b >= kf * grp
    lo_f = jnp.where(ok_f, jnp.maximum(lo_f_raw, -sn), -sn)
    lo_b = jnp.where(ok_b, jnp.maximum(lo_b_raw, -sn), -sn)

    nchunks = n_rows // _CH
    b_idx = jnp.minimum(cnt_i // _CH, nchunks - 1)
    rowid_ch = lax.broadcasted_iota(jnp.int32, (_CH, 1), 0)

    def it(_, st):
        lo_f, hi_f, lo_b, hi_b = st
        mid_f = 0.5 * (lo_f + hi_f)
        mid_b = 0.5 * (lo_b + hi_b)
        cf = jnp.zeros((1, _R), jnp.float32)
        cb = jnp.zeros((1, _R), jnp.float32)
        zero = jnp.zeros((1, _R), jnp.float32)
        for j in range(nchunks):
            thr = jnp.where(j <= b_idx, mid_f, mid_b)
            slab = sim[j * _CH:(j + 1) * _CH, :]
            part = jnp.sum((slab >= thr).astype(jnp.float32),
                           axis=0, keepdims=True)
            cf = cf + jnp.where(j < b_idx, part, zero)
            cb = cb + jnp.where(j > b_idx, part, zero)
        slab = sim_ref[pl.ds(b_idx * _CH, _CH), :]
        bpred = (rowid_ch + b_idx * _CH) < cnt_i
        gef = (slab >= mid_f) & bpred
        geb = (slab >= mid_b) & jnp.logical_not(bpred)
        cf = cf + jnp.sum(gef.astype(jnp.float32), axis=0, keepdims=True)
        cb = cb + jnp.sum(geb.astype(jnp.float32), axis=0, keepdims=True)
        pf = cf >= n_f
        pb = cb >= n_b
        lo_f = jnp.where(pf, mid_f, lo_f)
        hi_f = jnp.where(pf, hi_f, mid_f)
        lo_b = jnp.where(pb, mid_b, lo_b)
        hi_b = jnp.where(pb, hi_b, mid_b)
        return lo_f, hi_f, lo_b, hi_b

    lo_f, hi_f, lo_b, hi_b = jax.lax.fori_loop(
        0, _ITERS, it, (lo_f, hi_f, lo_b, hi_b))

    gtf = ((sim > lo_f) & infg).astype(jnp.float32)
    gtb = ((sim > lo_b) & jnp.logical_not(infg)).astype(jnp.float32)
    s_f = jnp.sum(gtf * sim, axis=0, keepdims=True)
    s_b = jnp.sum(gtb * sim, axis=0, keepdims=True)
    cgf = jnp.sum(gtf, axis=0, keepdims=True)
    cgb = jnp.sum(gtb, axis=0, keepdims=True)

    t_f = jnp.where(lo_f > jnp.float32(-1e38), lo_f, 0.0)
    t_b = jnp.where(lo_b > jnp.float32(-1e38), lo_b, 0.0)
    res_f = jnp.where(n_f > 0,
                      (s_f + (n_f - cgf) * t_f) / jnp.maximum(n_f, 1.0), 0.0)
    res_b = jnp.where(n_b > 0,
                      (s_b + (n_b - cgb) * t_b) / jnp.maximum(n_b, 1.0), 0.0)

    sn_inv = 1.0 / jnp.maximum(sn, 1e-12)
    ofg_ref[...] = (res_f * sn_inv).reshape(1, 1, _R)
    obg_ref[...] = (res_b * sn_inv).reshape(1, 1, _R)


@functools.partial(jax.jit, static_argnums=())
def kernel(query_label, color, q_feat, s_feat):
    Hq, Wq = int(q_feat.shape[2]), int(q_feat.shape[3])
    C = int(q_feat.shape[1])
    N = Hq * Wq
    Hs, Ws = int(s_feat.shape[2]), int(s_feat.shape[3])
    M = Hs * Ws

    feats = q_feat.reshape(C, N).T
    sfm = s_feat.reshape(C, M)

    mask = (query_label.reshape(N, 3) == color[None, :]).all(-1)
    perm = jnp.argsort(jnp.logical_not(mask)).astype(jnp.int32)
    cnt = jnp.sum(mask, dtype=jnp.int32).reshape(1, 1)

    fperm = _sc_gather(feats, perm)

    nblk = M // _R
    out_shape = jax.ShapeDtypeStruct((nblk, 1, _R), jnp.float32)
    ofg, obg = pl.pallas_call(
        _body,
        grid=(nblk,),
        in_specs=[
            pl.BlockSpec(memory_space=pltpu.SMEM),
            pl.BlockSpec((N, C), lambda i: (0, 0)),
            pl.BlockSpec((C, _R), lambda i: (0, i)),
        ],
        out_specs=[
            pl.BlockSpec((1, 1, _R), lambda i: (i, 0, 0)),
            pl.BlockSpec((1, 1, _R), lambda i: (i, 0, 0)),
        ],
        out_shape=[out_shape, out_shape],
        scratch_shapes=[pltpu.VMEM((N, C), jnp.float32),
                        pltpu.VMEM((N, _R), jnp.float32)],
    )(cnt, fperm, sfm)

    return (ofg.reshape(Hs, Ws), obg.reshape(Hs, Ws))
